# baseline (speedup 1.0000x reference)
import functools

import jax
import jax.numpy as jnp
from jax import lax
from jax.experimental import pallas as pl
from jax.experimental.pallas import tpu as pltpu
from jax.experimental.pallas import tpu_sc as plsc

E = 800000
D = 64
NROWS = 60
NC = 2
NS = 16
NW = NC * NS
C = 640
NCHUNKS = E // C
MAX_ITERS = (NCHUNKS + NW - 1) // NW
BG = 6400
NB = E // BG


def _lut_body(w0_ref, w1_ref, w2_ref, lut_ref):
    r = lax.broadcasted_iota(jnp.int32, (NROWS, 1), 0)
    a0 = (r // 12 == lax.broadcasted_iota(jnp.int32, (NROWS, 5), 1)).astype(jnp.float32)
    a1 = ((r % 12) // 2 == lax.broadcasted_iota(jnp.int32, (NROWS, 6), 1)).astype(jnp.float32)
    a2 = (r % 2 == lax.broadcasted_iota(jnp.int32, (NROWS, 2), 1)).astype(jnp.float32)
    f32 = jnp.float32
    lut_ref[...] = (
        jnp.dot(a0, w0_ref[...], preferred_element_type=f32)
        + jnp.dot(a1, w1_ref[...], preferred_element_type=f32)
        + jnp.dot(a2, w2_ref[...], preferred_element_type=f32)
    )


_build_lut = pl.pallas_call(
    _lut_body,
    out_shape=jax.ShapeDtypeStruct((NROWS, D), jnp.float32),
)


def _ridx_body(x_ref, r_ref):
    xb = x_ref[...].astype(jnp.float32)
    c0 = lax.broadcasted_iota(jnp.int32, (1, 3), 1)
    coef = jnp.where(c0 == 0, 12.0, jnp.where(c0 == 1, 2.0, 1.0)).astype(jnp.float32)
    r = lax.dot_general(coef, xb, (((1,), (1,)), ((), ())),
                        preferred_element_type=jnp.float32)
    r_ref[...] = r.astype(jnp.int32).reshape(1, 1, BG)


_fuse_ridx = pl.pallas_call(
    _ridx_body,
    grid=(NB,),
    in_specs=[pl.BlockSpec((BG, 3), lambda b: (b, 0))],
    out_specs=pl.BlockSpec((1, 1, BG), lambda b: (b, 0, 0)),
    out_shape=jax.ShapeDtypeStruct((NB, 1, BG), jnp.int32),
)

CPB = BG // C


@functools.cache
def _make_sc_lookup():
    @functools.partial(
        pl.kernel,
        out_type=jax.ShapeDtypeStruct((E, D), jnp.float32),
        mesh=plsc.VectorSubcoreMesh(
            core_axis_name="c", subcore_axis_name="s",
            num_cores=NC, num_subcores=NS,
        ),
        scratch_types=[
            pltpu.VMEM((NROWS, D), jnp.float32),
            pltpu.VMEM((C,), jnp.int32),
            pltpu.VMEM((C, D), jnp.float32),
        ],
        compiler_params=pltpu.CompilerParams(use_tc_tiling_on_sc=True),
    )
    def _sc_lookup(r_hbm, lut_hbm, out_hbm, lut_v, r_v, rows_v):
        w = lax.axis_index("s") * NC + lax.axis_index("c")
        pltpu.sync_copy(lut_hbm, lut_v)

        def chunk_body(i, carry):
            cid = w + NW * i

            @pl.when(cid < NCHUNKS)
            def _():
                base = pl.multiple_of(cid * C, 128)
                pltpu.sync_copy(
                    r_hbm.at[cid // CPB, 0, pl.ds((cid % CPB) * C, C)], r_v)

                @plsc.parallel_loop(0, C // 16, unroll=2)
                def edge_body(v):
                    rvec = r_v[pl.ds(v * 16, 16)]
                    for lane in range(16):
                        r = rvec[lane]
                        e = v * 16 + lane
                        for g in range(4):
                            sl = pl.ds(g * 16, 16)
                            rows_v[e, sl] = lut_v[r, sl]

                pltpu.sync_copy(rows_v, out_hbm.at[pl.ds(base, C)])

            return carry

        lax.fori_loop(0, MAX_ITERS, chunk_body, 0)

    return _sc_lookup


def kernel(x, W0, W1, W2):
    x = x.astype(jnp.int32)
    lut = _build_lut(W0, W1, W2)
    ridx = _fuse_ridx(x)
    return _make_sc_lookup()(ridx, lut)
